# baseline (speedup 1.0000x reference)
import jax
import jax.numpy as jnp
from jax.experimental import pallas as pl
from jax.experimental.pallas import tpu as pltpu


_CHUNKS = 4


def _copy_body(w_ref, out_ref, sems):
    batch = out_ref.shape[0]
    seq_len = w_ref.shape[0]
    chunk = seq_len // _CHUNKS
    copies = []
    for b in range(batch):
        for c in range(_CHUNKS):
            sl = pl.ds(c * chunk, chunk)
            copies.append(
                pltpu.make_async_copy(
                    w_ref.at[sl, :], out_ref.at[b, sl, :], sems.at[b, c]
                )
            )
    for cp in copies:
        cp.start()
    for cp in copies:
        cp.wait()


def kernel(tokens, W_pos):
    batch, seq_len = tokens.shape
    d_model = W_pos.shape[1]
    out = pl.pallas_call(
        _copy_body,
        in_specs=[pl.BlockSpec(memory_space=pl.ANY)],
        out_specs=pl.BlockSpec(memory_space=pl.ANY),
        out_shape=jax.ShapeDtypeStruct((batch, seq_len, d_model), W_pos.dtype),
        scratch_shapes=[pltpu.SemaphoreType.DMA((batch, _CHUNKS))],
    )(W_pos[:seq_len])
    return out


# speedup vs baseline: 75.0640x; 75.0640x over previous
import jax
import jax.numpy as jnp
from jax.experimental import pallas as pl
from jax.experimental.pallas import tpu as pltpu


_CHUNKS = 4


def _copy_body(w_ref, out_ref, vmem, in_sems, out_sems):
    batch = out_ref.shape[0]
    seq_len = w_ref.shape[0]
    chunk = seq_len // _CHUNKS
    ins = []
    for c in range(_CHUNKS):
        sl = pl.ds(c * chunk, chunk)
        cp = pltpu.make_async_copy(w_ref.at[sl, :], vmem.at[sl, :], in_sems.at[c])
        cp.start()
        ins.append(cp)
    outs = []
    for c in range(_CHUNKS):
        ins[c].wait()
        sl = pl.ds(c * chunk, chunk)
        for b in range(batch):
            cp = pltpu.make_async_copy(
                vmem.at[sl, :], out_ref.at[b, sl, :], out_sems.at[b, c]
            )
            cp.start()
            outs.append(cp)
    for cp in outs:
        cp.wait()


def kernel(tokens, W_pos):
    batch, seq_len = tokens.shape
    d_model = W_pos.shape[1]
    out = pl.pallas_call(
        _copy_body,
        in_specs=[pl.BlockSpec(memory_space=pl.ANY)],
        out_specs=pl.BlockSpec(memory_space=pl.ANY),
        out_shape=jax.ShapeDtypeStruct((batch, seq_len, d_model), W_pos.dtype),
        scratch_shapes=[
            pltpu.VMEM((seq_len, d_model), W_pos.dtype),
            pltpu.SemaphoreType.DMA((_CHUNKS,)),
            pltpu.SemaphoreType.DMA((batch, _CHUNKS)),
        ],
    )(W_pos[:seq_len])
    return out
